# baseline (speedup 1.0000x reference)
import functools

import jax
import jax.numpy as jnp
from jax import lax
from jax.experimental import pallas as pl
from jax.experimental.pallas import tpu as pltpu
from jax.experimental.pallas import tpu_sc as plsc

_N = 10000
_E = 320000
_NC = 2
_NP = 2
_NS = 16
_SPANS = ((0, 4992), (4992, 5008))
_AR = 5024
_TRASH = _AR - 1
_B = 128
_NBA = 160
_EP = _NS * _NBA * _B
_NBB = 88
_CAP = _NBB * _B
_RPS = 304
_MB = 1000
_GRID = _N // _MB


def _sc_mesh():
    return plsc.VectorSubcoreMesh(core_axis_name="c", subcore_axis_name="s")


def _stripe_copy(src, dst, s, rows, base_src=0, base_dst=0):
    row0 = s * _RPS
    pltpu.sync_copy(src.at[pl.ds(base_src + row0, _RPS)],
                    dst.at[pl.ds(base_dst + row0, _RPS)])
    tail0 = _RPS * _NS
    tail = rows - tail0

    @pl.when(s == _NS - 1)
    def _():
        pltpu.sync_copy(src.at[pl.ds(base_src + tail0, tail)],
                        dst.at[pl.ds(base_dst + tail0, tail)])


def _sc_bucket(src, dst, zero_l, trash_l):
    @functools.partial(
        pl.kernel,
        out_type=(
            jax.ShapeDtypeStruct((_NC, _NP, _NS, _CAP), jnp.int32),
            jax.ShapeDtypeStruct((_NC, _NP, _NS, _CAP), jnp.int32),
        ),
        mesh=_sc_mesh(),
        compiler_params=pltpu.CompilerParams(needs_layout_passes=False),
        scratch_types=[
            pltpu.VMEM((_NBA, _B), jnp.int32),
            pltpu.VMEM((_NBA, _B), jnp.int32),
            pltpu.VMEM((_CAP,), jnp.int32),
            pltpu.VMEM((_CAP,), jnp.int32),
            pltpu.VMEM((_CAP,), jnp.int32),
            pltpu.VMEM((_CAP,), jnp.int32),
        ],
    )
    def k(src_hbm, dst_hbm, zero_hbm, trash_hbm, srcL_hbm, dstL_hbm,
          src_v, dst_v, sA, dA, sB, dB):
        c = lax.axis_index("c")
        s = lax.axis_index("s")
        off = c * _N
        pltpu.sync_copy(src_hbm.at[s], src_v)
        pltpu.sync_copy(dst_hbm.at[s], dst_v)
        pltpu.sync_copy(zero_hbm, sA)
        pltpu.sync_copy(zero_hbm, sB)
        pltpu.sync_copy(trash_hbm, dA)
        pltpu.sync_copy(trash_hbm, dB)

        def chunk(i, carry):
            c0, c1 = carry
            r = i // 8
            kk = (i % 8) * 16
            dv = dst_v[r, pl.ds(kk, 16)]
            sv = src_v[r, pl.ds(kk, 16)] + off
            m0 = dv < _SPANS[0][1]
            cs0 = plsc.cumsum(m0.astype(jnp.int32))
            pos0 = c0 + cs0 - 1
            ok0 = m0 & (pos0 < _CAP)
            plsc.store_scatter(sA, [pos0], sv, mask=ok0)
            plsc.store_scatter(dA, [pos0], dv, mask=ok0)
            m1 = (dv >= _SPANS[1][0]) & (dv < _N)
            cs1 = plsc.cumsum(m1.astype(jnp.int32))
            pos1 = c1 + cs1 - 1
            ok1 = m1 & (pos1 < _CAP)
            plsc.store_scatter(sB, [pos1], sv, mask=ok1)
            plsc.store_scatter(dB, [pos1], dv - _SPANS[1][0], mask=ok1)
            return (c0 + jnp.max(cs0), c1 + jnp.max(cs1))

        lax.fori_loop(0, _NBA * 8, chunk, (0, 0))

        pltpu.sync_copy(sA, srcL_hbm.at[c].at[0].at[s])
        pltpu.sync_copy(dA, dstL_hbm.at[c].at[0].at[s])
        pltpu.sync_copy(sB, srcL_hbm.at[c].at[1].at[s])
        pltpu.sync_copy(dB, dstL_hbm.at[c].at[1].at[s])

    return k(src, dst, zero_l, trash_l)


def _sc_hist(dstL, ones, zeros):
    @functools.partial(
        pl.kernel,
        out_type=jax.ShapeDtypeStruct((_N, 128), jnp.float32),
        mesh=_sc_mesh(),
        scratch_types=[
            pltpu.VMEM((_NBB, _B), jnp.int32),
            pltpu.VMEM((_B, 128), jnp.float32),
            pltpu.VMEM_SHARED((_AR, 128), jnp.float32),
            pltpu.SemaphoreType.DMA,
        ],
    )
    def k(dstL_hbm, ones_hbm, zeros_hbm, out_hbm, dst_v, ones_v, acc, semS):
        c = lax.axis_index("c")
        s = lax.axis_index("s")
        _stripe_copy(zeros_hbm, acc, s, _AR)
        pltpu.sync_copy(ones_hbm, ones_v)
        pltpu.sync_copy(dstL_hbm.at[c].at[c].at[s], dst_v)
        plsc.subcore_barrier()

        @pl.loop(0, _NBB)
        def _(j):
            @pl.when(j >= 8)
            def _():
                pltpu.make_async_copy(ones_v, acc.at[dst_v.at[0]], semS).wait()

            pltpu.async_copy(ones_v, acc.at[dst_v.at[j]], semS, add=True)

        for _i in range(8):
            pltpu.make_async_copy(ones_v, acc.at[dst_v.at[0]], semS).wait()

        plsc.subcore_barrier()
        for cc, (pb, rows) in enumerate(_SPANS):
            @pl.when(c == cc)
            def _():
                _stripe_copy(acc, out_hbm, s, rows, base_dst=pb)

    return k(dstL, ones, zeros)


def _sc_agg(y, srcL, dstL):
    @functools.partial(
        pl.kernel,
        out_type=jax.ShapeDtypeStruct((_NC * _N, 128), jnp.float32),
        mesh=_sc_mesh(),
        scratch_types=[
            pltpu.VMEM((_NBB, _B), jnp.int32),
            pltpu.VMEM((_NBB, _B), jnp.int32),
            pltpu.VMEM_SHARED((_AR, 128), jnp.float32),
            pltpu.VMEM((_B, 128), jnp.float32),
            pltpu.VMEM((_B, 128), jnp.float32),
            pltpu.SemaphoreType.DMA,
            pltpu.SemaphoreType.DMA,
        ],
    )
    def k(y_hbm, srcL_hbm, dstL_hbm, out_hbm, src_v, dst_v, acc, g0, g1,
          semA, semB):
        c = lax.axis_index("c")
        s = lax.axis_index("s")
        for p, (pb, rows) in enumerate(_SPANS):
            base = c * _N + pb
            _stripe_copy(y_hbm, acc, s, rows, base_src=base)
            pltpu.sync_copy(srcL_hbm.at[c].at[p].at[s], src_v)
            pltpu.sync_copy(dstL_hbm.at[c].at[p].at[s], dst_v)
            plsc.subcore_barrier()

            pltpu.async_copy(y_hbm.at[src_v.at[0]], g0, semA)

            @pl.loop(0, _NBB, step=2)
            def _(j):
                pltpu.make_async_copy(y_hbm.at[src_v.at[j]], g0, semA).wait()
                pltpu.async_copy(y_hbm.at[src_v.at[j + 1]], g1, semB)
                pltpu.sync_copy(g0, acc.at[dst_v.at[j]], add=True)
                pltpu.make_async_copy(y_hbm.at[src_v.at[j + 1]], g1, semB).wait()

                @pl.when(j + 2 < _NBB)
                def _():
                    pltpu.async_copy(y_hbm.at[src_v.at[j + 2]], g0, semA)

                pltpu.sync_copy(g1, acc.at[dst_v.at[j + 1]], add=True)

            plsc.subcore_barrier()
            _stripe_copy(acc, out_hbm, s, rows, base_dst=base)

    return k(y, srcL, dstL)


def _dinv_of(hist_blk):
    deg = jnp.sum(hist_blk, axis=1) * (1.0 / 128.0) + 1.0
    return lax.rsqrt(deg)[:, None]


def _tc1_body(hist_ref, x_ref, w_ref, out_ref):
    dinv = _dinv_of(hist_ref[...])
    y = jnp.dot(x_ref[...], w_ref[...], preferred_element_type=jnp.float32) * dinv
    out_ref[0] = y[:, :128]
    out_ref[1] = y[:, 128:]


def _tc2_body(hist_ref, a_ref, b_ref, w_ref, out_ref):
    dinv = _dinv_of(hist_ref[...])
    a = jnp.concatenate([a_ref[0], a_ref[1]], axis=1)
    h = jnp.maximum(a * dinv + b_ref[...], 0.0)
    y = jnp.dot(h, w_ref[...], preferred_element_type=jnp.float32) * dinv
    out_ref[0] = y[:, :128]
    out_ref[1] = y[:, 128:]


def _tc3_body(hist_ref, a_ref, b2_ref, wl_ref, bl_ref, out_ref):
    dinv = _dinv_of(hist_ref[...])
    a = jnp.concatenate([a_ref[0], a_ref[1]], axis=1)
    h = jnp.maximum(a * dinv + b2_ref[...], 0.0)
    out_ref[...] = (
        jnp.dot(h, wl_ref[...], preferred_element_type=jnp.float32) + bl_ref[...]
    )


_HIST_SPEC = pl.BlockSpec((_MB, 128), lambda i: (i, 0))
_HALF_SPEC = pl.BlockSpec((_NC, _MB, 128), lambda i: (0, i, 0))


def _tc1(hist, x, W1):
    return pl.pallas_call(
        _tc1_body,
        grid=(_GRID,),
        in_specs=[
            _HIST_SPEC,
            pl.BlockSpec((_MB, 128), lambda i: (i, 0)),
            pl.BlockSpec((128, 256), lambda i: (0, 0)),
        ],
        out_specs=_HALF_SPEC,
        out_shape=jax.ShapeDtypeStruct((_NC, _N, 128), jnp.float32),
    )(hist, x, W1)


def _tc2(hist, agg, b1, W2):
    return pl.pallas_call(
        _tc2_body,
        grid=(_GRID,),
        in_specs=[
            _HIST_SPEC,
            _HALF_SPEC,
            pl.BlockSpec((1, 256), lambda i: (0, 0)),
            pl.BlockSpec((256, 256), lambda i: (0, 0)),
        ],
        out_specs=_HALF_SPEC,
        out_shape=jax.ShapeDtypeStruct((_NC, _N, 128), jnp.float32),
    )(hist, agg, b1, W2)


def _tc3(hist, agg, b2, Wl, bl):
    return pl.pallas_call(
        _tc3_body,
        grid=(_GRID,),
        in_specs=[
            _HIST_SPEC,
            _HALF_SPEC,
            pl.BlockSpec((1, 256), lambda i: (0, 0)),
            pl.BlockSpec((256, 128), lambda i: (0, 0)),
            pl.BlockSpec((1, 128), lambda i: (0, 0)),
        ],
        out_specs=pl.BlockSpec((_MB, 128), lambda i: (i, 0)),
        out_shape=jax.ShapeDtypeStruct((_N, 128), jnp.float32),
    )(hist, agg, b2, Wl, bl)


def kernel(x, edge_index, W1, b1, W2, b2, Wl, bl):
    src = edge_index[0].astype(jnp.int32)
    dst = edge_index[1].astype(jnp.int32)
    pad = _EP - _E
    src = jnp.concatenate([src, jnp.zeros((pad,), jnp.int32)])
    dst = jnp.concatenate([dst, jnp.full((pad,), _N, jnp.int32)])
    srcr = src.reshape(_NS, _NBA, _B)
    dstr = dst.reshape(_NS, _NBA, _B)
    zero_l = jnp.zeros((_CAP,), jnp.int32)
    trash_l = jnp.full((_CAP,), _TRASH, jnp.int32)
    ones = jnp.ones((_B, 128), jnp.float32)
    zeros = jnp.zeros((_AR, 128), jnp.float32)

    srcL, dstL = _sc_bucket(srcr, dstr, zero_l, trash_l)
    srcL = srcL.reshape(_NC, _NP, _NS, _NBB, _B)
    dstL = dstL.reshape(_NC, _NP, _NS, _NBB, _B)

    hist = _sc_hist(dstL, ones, zeros)
    y1 = _tc1(hist, x, W1)
    agg1 = _sc_agg(y1.reshape(_NC * _N, 128), srcL, dstL).reshape(_NC, _N, 128)
    y2 = _tc2(hist, agg1, b1.reshape(1, 256), W2)
    agg2 = _sc_agg(y2.reshape(_NC * _N, 128), srcL, dstL).reshape(_NC, _N, 128)
    return _tc3(hist, agg2, b2.reshape(1, 256), Wl, bl.reshape(1, 128))


# speedup vs baseline: 8.8225x; 8.8225x over previous
import functools

import jax
import jax.numpy as jnp
from jax import lax
from jax.experimental import pallas as pl
from jax.experimental.pallas import tpu as pltpu
from jax.experimental.pallas import tpu_sc as plsc

_N = 10000
_E = 320000
_NC = 2
_NP = 2
_NS = 16
_SPANS = ((0, 4992), (4992, 5008))
_AR = 5024
_TRASH = _AR - 1
_B = 128
_NBA = 160
_EP = _NS * _NBA * _B
_NBB = 88
_CAP = _NBB * _B
_RPS = 304
_MB = 1000
_GRID = _N // _MB


def _sc_mesh():
    return plsc.VectorSubcoreMesh(core_axis_name="c", subcore_axis_name="s")


def _stripe_copy(src, dst, s, rows, base_src=0, base_dst=0):
    row0 = s * _RPS
    pltpu.sync_copy(src.at[pl.ds(base_src + row0, _RPS)],
                    dst.at[pl.ds(base_dst + row0, _RPS)])
    tail0 = _RPS * _NS
    tail = rows - tail0

    @pl.when(s == _NS - 1)
    def _():
        pltpu.sync_copy(src.at[pl.ds(base_src + tail0, tail)],
                        dst.at[pl.ds(base_dst + tail0, tail)])


def _sc_bucket(src, dst, zero_l, trash_l):
    @functools.partial(
        pl.kernel,
        out_type=(
            jax.ShapeDtypeStruct((_NC, _NP, _NS, _CAP), jnp.int32),
            jax.ShapeDtypeStruct((_NC, _NP, _NS, _CAP), jnp.int32),
        ),
        mesh=_sc_mesh(),
        compiler_params=pltpu.CompilerParams(needs_layout_passes=False),
        scratch_types=[
            pltpu.VMEM((_NBA, _B), jnp.int32),
            pltpu.VMEM((_NBA, _B), jnp.int32),
            pltpu.VMEM((_CAP,), jnp.int32),
            pltpu.VMEM((_CAP,), jnp.int32),
            pltpu.VMEM((_CAP,), jnp.int32),
            pltpu.VMEM((_CAP,), jnp.int32),
        ],
    )
    def k(src_hbm, dst_hbm, zero_hbm, trash_hbm, srcL_hbm, dstL_hbm,
          src_v, dst_v, sA, dA, sB, dB):
        c = lax.axis_index("c")
        s = lax.axis_index("s")
        off = c * _N
        pltpu.sync_copy(src_hbm.at[s], src_v)
        pltpu.sync_copy(dst_hbm.at[s], dst_v)
        pltpu.sync_copy(zero_hbm, sA)
        pltpu.sync_copy(zero_hbm, sB)
        pltpu.sync_copy(trash_hbm, dA)
        pltpu.sync_copy(trash_hbm, dB)

        def chunk(i, carry):
            c0, c1 = carry
            r = i // 8
            kk = (i % 8) * 16
            dv = dst_v[r, pl.ds(kk, 16)]
            sv = src_v[r, pl.ds(kk, 16)] + off
            m0 = dv < _SPANS[0][1]
            cs0 = plsc.cumsum(m0.astype(jnp.int32))
            pos0 = c0 + cs0 - 1
            ok0 = m0 & (pos0 < _CAP)
            plsc.store_scatter(sA, [pos0], sv, mask=ok0)
            plsc.store_scatter(dA, [pos0], dv, mask=ok0)
            m1 = (dv >= _SPANS[1][0]) & (dv < _N)
            cs1 = plsc.cumsum(m1.astype(jnp.int32))
            pos1 = c1 + cs1 - 1
            ok1 = m1 & (pos1 < _CAP)
            plsc.store_scatter(sB, [pos1], sv, mask=ok1)
            plsc.store_scatter(dB, [pos1], dv - _SPANS[1][0], mask=ok1)
            return (c0 + jnp.max(cs0), c1 + jnp.max(cs1))

        lax.fori_loop(0, _NBA * 8, chunk, (0, 0))

        pltpu.sync_copy(sA, srcL_hbm.at[c].at[0].at[s])
        pltpu.sync_copy(dA, dstL_hbm.at[c].at[0].at[s])
        pltpu.sync_copy(sB, srcL_hbm.at[c].at[1].at[s])
        pltpu.sync_copy(dB, dstL_hbm.at[c].at[1].at[s])

    return k(src, dst, zero_l, trash_l)


def _sc_hist(dstL, ones, zeros):
    @functools.partial(
        pl.kernel,
        out_type=jax.ShapeDtypeStruct((_N, 128), jnp.float32),
        mesh=_sc_mesh(),
        scratch_types=[
            pltpu.VMEM((_NBB, _B), jnp.int32),
            pltpu.VMEM((_B, 128), jnp.float32),
            pltpu.VMEM_SHARED((_AR, 128), jnp.float32),
            pltpu.SemaphoreType.DMA,
        ],
    )
    def k(dstL_hbm, ones_hbm, zeros_hbm, out_hbm, dst_v, ones_v, acc, semS):
        c = lax.axis_index("c")
        s = lax.axis_index("s")
        _stripe_copy(zeros_hbm, acc, s, _AR)
        pltpu.sync_copy(ones_hbm, ones_v)
        pltpu.sync_copy(dstL_hbm.at[c].at[c].at[s], dst_v)
        plsc.subcore_barrier()

        @pl.loop(0, _NBB)
        def _(j):
            @pl.when(j >= 8)
            def _():
                pltpu.make_async_copy(ones_v, acc.at[dst_v.at[0]], semS).wait()

            pltpu.async_copy(ones_v, acc.at[dst_v.at[j]], semS, add=True)

        for _i in range(8):
            pltpu.make_async_copy(ones_v, acc.at[dst_v.at[0]], semS).wait()

        plsc.subcore_barrier()
        for cc, (pb, rows) in enumerate(_SPANS):
            @pl.when(c == cc)
            def _():
                _stripe_copy(acc, out_hbm, s, rows, base_dst=pb)

    return k(dstL, ones, zeros)


def _sc_agg(y, srcL, dstL):
    @functools.partial(
        pl.kernel,
        out_type=jax.ShapeDtypeStruct((_NC * _N, 128), jnp.float32),
        mesh=_sc_mesh(),
        scratch_types=[
            pltpu.VMEM((_NBB, _B), jnp.int32),
            pltpu.VMEM((_NBB, _B), jnp.int32),
            pltpu.VMEM_SHARED((_AR, 128), jnp.float32),
            pltpu.VMEM((_B, 128), jnp.float32),
            pltpu.VMEM((_B, 128), jnp.float32),
            pltpu.SemaphoreType.DMA,
            pltpu.SemaphoreType.DMA,
        ],
    )
    def k(y_hbm, srcL_hbm, dstL_hbm, out_hbm, src_v, dst_v, acc, g0, g1,
          semA, semB):
        c = lax.axis_index("c")
        s = lax.axis_index("s")
        for p, (pb, rows) in enumerate(_SPANS):
            base = c * _N + pb
            _stripe_copy(y_hbm, acc, s, rows, base_src=base)
            pltpu.sync_copy(srcL_hbm.at[c].at[p].at[s], src_v)
            pltpu.sync_copy(dstL_hbm.at[c].at[p].at[s], dst_v)
            plsc.subcore_barrier()

            pltpu.async_copy(y_hbm.at[src_v.at[0]], g0, semA)

            @pl.loop(0, _NBB, step=2)
            def _(j):
                pltpu.make_async_copy(y_hbm.at[src_v.at[j]], g0, semA).wait()
                pltpu.async_copy(y_hbm.at[src_v.at[j + 1]], g1, semB)
                pltpu.sync_copy(g0, acc.at[dst_v.at[j]], add=True)
                pltpu.make_async_copy(y_hbm.at[src_v.at[j + 1]], g1, semB).wait()

                @pl.when(j + 2 < _NBB)
                def _():
                    pltpu.async_copy(y_hbm.at[src_v.at[j + 2]], g0, semA)

                pltpu.sync_copy(g1, acc.at[dst_v.at[j + 1]], add=True)

            plsc.subcore_barrier()
            _stripe_copy(acc, out_hbm, s, rows, base_dst=base)

    return k(y, srcL, dstL)


def _dinv_of(hist_blk):
    deg = jnp.sum(hist_blk, axis=1) * (1.0 / 128.0) + 1.0
    return lax.rsqrt(deg)[:, None]


def _tc1_body(hist_ref, x_ref, w_ref, out_ref):
    dinv = _dinv_of(hist_ref[...])
    y = jnp.dot(x_ref[...], w_ref[...], preferred_element_type=jnp.float32) * dinv
    out_ref[0] = y[:, :128]
    out_ref[1] = y[:, 128:]


def _tc2_body(hist_ref, a_ref, b_ref, w_ref, out_ref):
    dinv = _dinv_of(hist_ref[...])
    a = jnp.concatenate([a_ref[0], a_ref[1]], axis=1)
    h = jnp.maximum(a * dinv + b_ref[...], 0.0)
    y = jnp.dot(h, w_ref[...], preferred_element_type=jnp.float32) * dinv
    out_ref[0] = y[:, :128]
    out_ref[1] = y[:, 128:]


def _tc3_body(hist_ref, a_ref, b2_ref, wl_ref, bl_ref, out_ref):
    dinv = _dinv_of(hist_ref[...])
    a = jnp.concatenate([a_ref[0], a_ref[1]], axis=1)
    h = jnp.maximum(a * dinv + b2_ref[...], 0.0)
    out_ref[...] = (
        jnp.dot(h, wl_ref[...], preferred_element_type=jnp.float32) + bl_ref[...]
    )


_HIST_SPEC = pl.BlockSpec((_MB, 128), lambda i: (i, 0))
_HALF_SPEC = pl.BlockSpec((_NC, _MB, 128), lambda i: (0, i, 0))


def _tc1(hist, x, W1):
    return pl.pallas_call(
        _tc1_body,
        grid=(_GRID,),
        in_specs=[
            _HIST_SPEC,
            pl.BlockSpec((_MB, 128), lambda i: (i, 0)),
            pl.BlockSpec((128, 256), lambda i: (0, 0)),
        ],
        out_specs=_HALF_SPEC,
        out_shape=jax.ShapeDtypeStruct((_NC, _N, 128), jnp.float32),
    )(hist, x, W1)


def _tc2(hist, agg, b1, W2):
    return pl.pallas_call(
        _tc2_body,
        grid=(_GRID,),
        in_specs=[
            _HIST_SPEC,
            _HALF_SPEC,
            pl.BlockSpec((1, 256), lambda i: (0, 0)),
            pl.BlockSpec((256, 256), lambda i: (0, 0)),
        ],
        out_specs=_HALF_SPEC,
        out_shape=jax.ShapeDtypeStruct((_NC, _N, 128), jnp.float32),
    )(hist, agg, b1, W2)


def _tc3(hist, agg, b2, Wl, bl):
    return pl.pallas_call(
        _tc3_body,
        grid=(_GRID,),
        in_specs=[
            _HIST_SPEC,
            _HALF_SPEC,
            pl.BlockSpec((1, 256), lambda i: (0, 0)),
            pl.BlockSpec((256, 128), lambda i: (0, 0)),
            pl.BlockSpec((1, 128), lambda i: (0, 0)),
        ],
        out_specs=pl.BlockSpec((_MB, 128), lambda i: (i, 0)),
        out_shape=jax.ShapeDtypeStruct((_N, 128), jnp.float32),
    )(hist, agg, b2, Wl, bl)


def kernel(x, edge_index, W1, b1, W2, b2, Wl, bl):
    src = edge_index[0].astype(jnp.int32)
    dst = edge_index[1].astype(jnp.int32)
    pad = _EP - _E
    src = jnp.concatenate([src, jnp.zeros((pad,), jnp.int32)])
    dst = jnp.concatenate([dst, jnp.full((pad,), _N, jnp.int32)])
    srcr = src.reshape(_NS, _NBA, _B)
    dstr = dst.reshape(_NS, _NBA, _B)
    zero_l = jnp.arange(_CAP, dtype=jnp.int32) % _N
    trash_l = jnp.full((_CAP,), _TRASH, jnp.int32)
    ones = jnp.ones((_B, 128), jnp.float32)
    zeros = jnp.zeros((_AR, 128), jnp.float32)

    srcL, dstL = _sc_bucket(srcr, dstr, zero_l, trash_l)
    srcL = srcL.reshape(_NC, _NP, _NS, _NBB, _B)
    dstL = dstL.reshape(_NC, _NP, _NS, _NBB, _B)

    hist = _sc_hist(dstL, ones, zeros)
    y1 = _tc1(hist, x, W1)
    agg1 = _sc_agg(y1.reshape(_NC * _N, 128), srcL, dstL).reshape(_NC, _N, 128)
    y2 = _tc2(hist, agg1, b1.reshape(1, 256), W2)
    agg2 = _sc_agg(y2.reshape(_NC * _N, 128), srcL, dstL).reshape(_NC, _N, 128)
    return _tc3(hist, agg2, b2.reshape(1, 256), Wl, bl.reshape(1, 128))


# speedup vs baseline: 9.3885x; 1.0641x over previous
import functools

import jax
import jax.numpy as jnp
from jax import lax
from jax.experimental import pallas as pl
from jax.experimental.pallas import tpu as pltpu
from jax.experimental.pallas import tpu_sc as plsc

_N = 10000
_E = 320000
_NC = 2
_NP = 2
_NS = 16
_SPANS = ((0, 4992), (4992, 5008))
_AR = 5024
_TRASH = _AR - 1
_B = 128
_NBA = 160
_EP = _NS * _NBA * _B
_NBB = 88
_CAP = _NBB * _B
_RPS = 304
_MB = 1000
_GRID = _N // _MB


def _sc_mesh():
    return plsc.VectorSubcoreMesh(core_axis_name="c", subcore_axis_name="s")


def _stripe_copy(src, dst, s, rows, base_src=0, base_dst=0):
    row0 = s * _RPS
    pltpu.sync_copy(src.at[pl.ds(base_src + row0, _RPS)],
                    dst.at[pl.ds(base_dst + row0, _RPS)])
    tail0 = _RPS * _NS
    tail = rows - tail0

    @pl.when(s == _NS - 1)
    def _():
        pltpu.sync_copy(src.at[pl.ds(base_src + tail0, tail)],
                        dst.at[pl.ds(base_dst + tail0, tail)])


def _sc_bucket(src, dst, zero_l, trash_l):
    @functools.partial(
        pl.kernel,
        out_type=(
            jax.ShapeDtypeStruct((_NC, _NP, _NS, _CAP), jnp.int32),
            jax.ShapeDtypeStruct((_NC, _NP, _NS, _CAP), jnp.int32),
        ),
        mesh=_sc_mesh(),
        compiler_params=pltpu.CompilerParams(needs_layout_passes=False),
        scratch_types=[
            pltpu.VMEM((_NBA, _B), jnp.int32),
            pltpu.VMEM((_NBA, _B), jnp.int32),
            pltpu.VMEM((_CAP,), jnp.int32),
            pltpu.VMEM((_CAP,), jnp.int32),
            pltpu.VMEM((_CAP,), jnp.int32),
            pltpu.VMEM((_CAP,), jnp.int32),
        ],
    )
    def k(src_hbm, dst_hbm, zero_hbm, trash_hbm, srcL_hbm, dstL_hbm,
          src_v, dst_v, sA, dA, sB, dB):
        c = lax.axis_index("c")
        s = lax.axis_index("s")
        off = c * _N
        pltpu.sync_copy(src_hbm.at[s], src_v)
        pltpu.sync_copy(dst_hbm.at[s], dst_v)
        pltpu.sync_copy(zero_hbm, sA)
        pltpu.sync_copy(zero_hbm, sB)
        pltpu.sync_copy(trash_hbm, dA)
        pltpu.sync_copy(trash_hbm, dB)

        def chunk(i, carry):
            c0, c1 = carry
            r = i // 8
            kk = (i % 8) * 16
            dv = dst_v[r, pl.ds(kk, 16)]
            sv = src_v[r, pl.ds(kk, 16)] + off
            m0 = dv < _SPANS[0][1]
            cs0 = plsc.cumsum(m0.astype(jnp.int32))
            pos0 = c0 + cs0 - 1
            ok0 = m0 & (pos0 < _CAP)
            plsc.store_scatter(sA, [pos0], sv, mask=ok0)
            plsc.store_scatter(dA, [pos0], dv, mask=ok0)
            m1 = (dv >= _SPANS[1][0]) & (dv < _N)
            cs1 = plsc.cumsum(m1.astype(jnp.int32))
            pos1 = c1 + cs1 - 1
            ok1 = m1 & (pos1 < _CAP)
            plsc.store_scatter(sB, [pos1], sv, mask=ok1)
            plsc.store_scatter(dB, [pos1], dv - _SPANS[1][0], mask=ok1)
            return (c0 + jnp.max(cs0), c1 + jnp.max(cs1))

        lax.fori_loop(0, _NBA * 8, chunk, (0, 0))

        pltpu.sync_copy(sA, srcL_hbm.at[c].at[0].at[s])
        pltpu.sync_copy(dA, dstL_hbm.at[c].at[0].at[s])
        pltpu.sync_copy(sB, srcL_hbm.at[c].at[1].at[s])
        pltpu.sync_copy(dB, dstL_hbm.at[c].at[1].at[s])

    return k(src, dst, zero_l, trash_l)


def _sc_hist(dstL, ones, zeros):
    @functools.partial(
        pl.kernel,
        out_type=jax.ShapeDtypeStruct((_N, 128), jnp.float32),
        mesh=_sc_mesh(),
        scratch_types=[
            pltpu.VMEM((_NBB, _B), jnp.int32),
            pltpu.VMEM((_B, 128), jnp.float32),
            pltpu.VMEM_SHARED((_AR, 128), jnp.float32),
            pltpu.SemaphoreType.DMA,
        ],
    )
    def k(dstL_hbm, ones_hbm, zeros_hbm, out_hbm, dst_v, ones_v, acc, semS):
        c = lax.axis_index("c")
        s = lax.axis_index("s")
        _stripe_copy(zeros_hbm, acc, s, _AR)
        pltpu.sync_copy(ones_hbm, ones_v)
        pltpu.sync_copy(dstL_hbm.at[c].at[c].at[s], dst_v)
        plsc.subcore_barrier()

        @pl.loop(0, _NBB)
        def _(j):
            @pl.when(j >= 8)
            def _():
                pltpu.make_async_copy(ones_v, acc.at[dst_v.at[0]], semS).wait()

            pltpu.async_copy(ones_v, acc.at[dst_v.at[j]], semS, add=True)

        for _i in range(8):
            pltpu.make_async_copy(ones_v, acc.at[dst_v.at[0]], semS).wait()

        plsc.subcore_barrier()
        for cc, (pb, rows) in enumerate(_SPANS):
            @pl.when(c == cc)
            def _():
                _stripe_copy(acc, out_hbm, s, rows, base_dst=pb)

    return k(dstL, ones, zeros)


def _sc_agg(y, srcL, dstL):
    @functools.partial(
        pl.kernel,
        out_type=jax.ShapeDtypeStruct((_NC * _N, 128), jnp.float32),
        mesh=_sc_mesh(),
        scratch_types=[
            pltpu.VMEM((_NBB, _B), jnp.int32),
            pltpu.VMEM((_NBB, _B), jnp.int32),
            pltpu.VMEM_SHARED((_AR, 128), jnp.float32),
        ] + [pltpu.VMEM((_B, 128), jnp.float32) for _ in range(3)]
          + [pltpu.SemaphoreType.DMA] * 6,
    )
    def k(y_hbm, srcL_hbm, dstL_hbm, out_hbm, src_v, dst_v, acc, *rest):
        g = rest[:3]
        semG = rest[3:6]
        semS = rest[6:9]
        c = lax.axis_index("c")
        s = lax.axis_index("s")
        for p, (pb, rows) in enumerate(_SPANS):
            base = c * _N + pb
            _stripe_copy(y_hbm, acc, s, rows, base_src=base)
            pltpu.sync_copy(srcL_hbm.at[c].at[p].at[s], src_v)
            pltpu.sync_copy(dstL_hbm.at[c].at[p].at[s], dst_v)
            plsc.subcore_barrier()

            for k_ in range(3):
                pltpu.async_copy(y_hbm.at[src_v.at[k_]], g[k_], semG[k_])

            @pl.loop(0, _NBB - 1, step=3)
            def _(j):
                for k_ in range(3):
                    pltpu.make_async_copy(
                        y_hbm.at[src_v.at[j + k_]], g[k_], semG[k_]).wait()
                    pltpu.async_copy(
                        g[k_], acc.at[dst_v.at[j + k_]], semS[k_], add=True)

                for k_ in range(3):
                    @pl.when(j + 3 + k_ < _NBB)
                    def _():
                        pltpu.make_async_copy(
                            g[k_], acc.at[dst_v.at[0]], semS[k_]).wait()
                        pltpu.async_copy(
                            y_hbm.at[src_v.at[j + 3 + k_]], g[k_], semG[k_])

            pltpu.make_async_copy(
                y_hbm.at[src_v.at[_NBB - 1]], g[0], semG[0]).wait()
            pltpu.async_copy(g[0], acc.at[dst_v.at[_NBB - 1]], semS[0],
                             add=True)
            for k_ in range(3):
                pltpu.make_async_copy(g[k_], acc.at[dst_v.at[0]], semS[k_]).wait()

            plsc.subcore_barrier()
            _stripe_copy(acc, out_hbm, s, rows, base_dst=base)

    return k(y, srcL, dstL)


def _dinv_of(hist_blk):
    deg = jnp.sum(hist_blk, axis=1) * (1.0 / 128.0) + 1.0
    return lax.rsqrt(deg)[:, None]


def _tc1_body(hist_ref, x_ref, w_ref, out_ref):
    dinv = _dinv_of(hist_ref[...])
    y = jnp.dot(x_ref[...], w_ref[...], preferred_element_type=jnp.float32) * dinv
    out_ref[0] = y[:, :128]
    out_ref[1] = y[:, 128:]


def _tc2_body(hist_ref, a_ref, b_ref, w_ref, out_ref):
    dinv = _dinv_of(hist_ref[...])
    a = jnp.concatenate([a_ref[0], a_ref[1]], axis=1)
    h = jnp.maximum(a * dinv + b_ref[...], 0.0)
    y = jnp.dot(h, w_ref[...], preferred_element_type=jnp.float32) * dinv
    out_ref[0] = y[:, :128]
    out_ref[1] = y[:, 128:]


def _tc3_body(hist_ref, a_ref, b2_ref, wl_ref, bl_ref, out_ref):
    dinv = _dinv_of(hist_ref[...])
    a = jnp.concatenate([a_ref[0], a_ref[1]], axis=1)
    h = jnp.maximum(a * dinv + b2_ref[...], 0.0)
    out_ref[...] = (
        jnp.dot(h, wl_ref[...], preferred_element_type=jnp.float32) + bl_ref[...]
    )


_HIST_SPEC = pl.BlockSpec((_MB, 128), lambda i: (i, 0))
_HALF_SPEC = pl.BlockSpec((_NC, _MB, 128), lambda i: (0, i, 0))


def _tc1(hist, x, W1):
    return pl.pallas_call(
        _tc1_body,
        grid=(_GRID,),
        in_specs=[
            _HIST_SPEC,
            pl.BlockSpec((_MB, 128), lambda i: (i, 0)),
            pl.BlockSpec((128, 256), lambda i: (0, 0)),
        ],
        out_specs=_HALF_SPEC,
        out_shape=jax.ShapeDtypeStruct((_NC, _N, 128), jnp.float32),
    )(hist, x, W1)


def _tc2(hist, agg, b1, W2):
    return pl.pallas_call(
        _tc2_body,
        grid=(_GRID,),
        in_specs=[
            _HIST_SPEC,
            _HALF_SPEC,
            pl.BlockSpec((1, 256), lambda i: (0, 0)),
            pl.BlockSpec((256, 256), lambda i: (0, 0)),
        ],
        out_specs=_HALF_SPEC,
        out_shape=jax.ShapeDtypeStruct((_NC, _N, 128), jnp.float32),
    )(hist, agg, b1, W2)


def _tc3(hist, agg, b2, Wl, bl):
    return pl.pallas_call(
        _tc3_body,
        grid=(_GRID,),
        in_specs=[
            _HIST_SPEC,
            _HALF_SPEC,
            pl.BlockSpec((1, 256), lambda i: (0, 0)),
            pl.BlockSpec((256, 128), lambda i: (0, 0)),
            pl.BlockSpec((1, 128), lambda i: (0, 0)),
        ],
        out_specs=pl.BlockSpec((_MB, 128), lambda i: (i, 0)),
        out_shape=jax.ShapeDtypeStruct((_N, 128), jnp.float32),
    )(hist, agg, b2, Wl, bl)


def kernel(x, edge_index, W1, b1, W2, b2, Wl, bl):
    src = edge_index[0].astype(jnp.int32)
    dst = edge_index[1].astype(jnp.int32)
    pad = _EP - _E
    src = jnp.concatenate([src, jnp.zeros((pad,), jnp.int32)])
    dst = jnp.concatenate([dst, jnp.full((pad,), _N, jnp.int32)])
    srcr = src.reshape(_NS, _NBA, _B)
    dstr = dst.reshape(_NS, _NBA, _B)
    zero_l = jnp.arange(_CAP, dtype=jnp.int32) % _N
    trash_l = jnp.full((_CAP,), _TRASH, jnp.int32)
    ones = jnp.ones((_B, 128), jnp.float32)
    zeros = jnp.zeros((_AR, 128), jnp.float32)

    srcL, dstL = _sc_bucket(srcr, dstr, zero_l, trash_l)
    srcL = srcL.reshape(_NC, _NP, _NS, _NBB, _B)
    dstL = dstL.reshape(_NC, _NP, _NS, _NBB, _B)

    hist = _sc_hist(dstL, ones, zeros)
    y1 = _tc1(hist, x, W1)
    agg1 = _sc_agg(y1.reshape(_NC * _N, 128), srcL, dstL).reshape(_NC, _N, 128)
    y2 = _tc2(hist, agg1, b1.reshape(1, 256), W2)
    agg2 = _sc_agg(y2.reshape(_NC * _N, 128), srcL, dstL).reshape(_NC, _N, 128)
    return _tc3(hist, agg2, b2.reshape(1, 256), Wl, bl.reshape(1, 128))


# speedup vs baseline: 9.9521x; 1.0600x over previous
import functools

import jax
import jax.numpy as jnp
from jax import lax
from jax.experimental import pallas as pl
from jax.experimental.pallas import tpu as pltpu
from jax.experimental.pallas import tpu_sc as plsc

_N = 10000
_E = 320000
_NC = 2
_NP = 2
_NS = 16
_SPANS = ((0, 4992), (4992, 5008))
_AR = 5024
_TRASH = _AR - 1
_B = 128
_NBA = 160
_EP = _NS * _NBA * _B
_NBB = 84
_CAP = _NBB * _B
_RPS = 304
_MB = 1000
_GRID = _N // _MB


def _sc_mesh():
    return plsc.VectorSubcoreMesh(core_axis_name="c", subcore_axis_name="s")


def _stripe_copy(src, dst, s, rows, base_src=0, base_dst=0):
    row0 = s * _RPS
    pltpu.sync_copy(src.at[pl.ds(base_src + row0, _RPS)],
                    dst.at[pl.ds(base_dst + row0, _RPS)])
    tail0 = _RPS * _NS
    tail = rows - tail0

    @pl.when(s == _NS - 1)
    def _():
        pltpu.sync_copy(src.at[pl.ds(base_src + tail0, tail)],
                        dst.at[pl.ds(base_dst + tail0, tail)])


def _sc_bucket(src, dst, zero_l, trash_l):
    @functools.partial(
        pl.kernel,
        out_type=(
            jax.ShapeDtypeStruct((_NC, _NP, _NS, _CAP), jnp.int32),
            jax.ShapeDtypeStruct((_NC, _NP, _NS, _CAP), jnp.int32),
        ),
        mesh=_sc_mesh(),
        compiler_params=pltpu.CompilerParams(needs_layout_passes=False),
        scratch_types=[
            pltpu.VMEM((_NBA, _B), jnp.int32),
            pltpu.VMEM((_NBA, _B), jnp.int32),
            pltpu.VMEM((_CAP,), jnp.int32),
            pltpu.VMEM((_CAP,), jnp.int32),
            pltpu.VMEM((_CAP,), jnp.int32),
            pltpu.VMEM((_CAP,), jnp.int32),
        ],
    )
    def k(src_hbm, dst_hbm, zero_hbm, trash_hbm, srcL_hbm, dstL_hbm,
          src_v, dst_v, sA, dA, sB, dB):
        c = lax.axis_index("c")
        s = lax.axis_index("s")
        off = c * _N
        pltpu.sync_copy(src_hbm.at[s], src_v)
        pltpu.sync_copy(dst_hbm.at[s], dst_v)
        pltpu.sync_copy(zero_hbm, sA)
        pltpu.sync_copy(zero_hbm, sB)
        pltpu.sync_copy(trash_hbm, dA)
        pltpu.sync_copy(trash_hbm, dB)

        def chunk(i, carry):
            c0, c1 = carry
            r = i // 8
            kk = (i % 8) * 16
            dv = dst_v[r, pl.ds(kk, 16)]
            sv = src_v[r, pl.ds(kk, 16)] + off
            m0 = dv < _SPANS[0][1]
            cs0 = plsc.cumsum(m0.astype(jnp.int32))
            pos0 = c0 + cs0 - 1
            ok0 = m0 & (pos0 < _CAP)
            plsc.store_scatter(sA, [pos0], sv, mask=ok0)
            plsc.store_scatter(dA, [pos0], dv, mask=ok0)
            m1 = (dv >= _SPANS[1][0]) & (dv < _N)
            cs1 = plsc.cumsum(m1.astype(jnp.int32))
            pos1 = c1 + cs1 - 1
            ok1 = m1 & (pos1 < _CAP)
            plsc.store_scatter(sB, [pos1], sv, mask=ok1)
            plsc.store_scatter(dB, [pos1], dv - _SPANS[1][0], mask=ok1)
            return (c0 + jnp.max(cs0), c1 + jnp.max(cs1))

        lax.fori_loop(0, _NBA * 8, chunk, (0, 0))

        pltpu.sync_copy(sA, srcL_hbm.at[c].at[0].at[s])
        pltpu.sync_copy(dA, dstL_hbm.at[c].at[0].at[s])
        pltpu.sync_copy(sB, srcL_hbm.at[c].at[1].at[s])
        pltpu.sync_copy(dB, dstL_hbm.at[c].at[1].at[s])

    return k(src, dst, zero_l, trash_l)


def _sc_hist(dstL, ones, zeros):
    @functools.partial(
        pl.kernel,
        out_type=jax.ShapeDtypeStruct((_N, 128), jnp.float32),
        mesh=_sc_mesh(),
        scratch_types=[
            pltpu.VMEM((_NBB, _B), jnp.int32),
            pltpu.VMEM((_B, 128), jnp.float32),
            pltpu.VMEM_SHARED((_AR, 128), jnp.float32),
            pltpu.SemaphoreType.DMA,
        ],
    )
    def k(dstL_hbm, ones_hbm, zeros_hbm, out_hbm, dst_v, ones_v, acc, semS):
        c = lax.axis_index("c")
        s = lax.axis_index("s")
        _stripe_copy(zeros_hbm, acc, s, _AR)
        pltpu.sync_copy(ones_hbm, ones_v)
        pltpu.sync_copy(dstL_hbm.at[c].at[c].at[s], dst_v)
        plsc.subcore_barrier()

        @pl.loop(0, _NBB)
        def _(j):
            @pl.when(j >= 8)
            def _():
                pltpu.make_async_copy(ones_v, acc.at[dst_v.at[0]], semS).wait()

            pltpu.async_copy(ones_v, acc.at[dst_v.at[j]], semS, add=True)

        for _i in range(8):
            pltpu.make_async_copy(ones_v, acc.at[dst_v.at[0]], semS).wait()

        plsc.subcore_barrier()
        for cc, (pb, rows) in enumerate(_SPANS):
            @pl.when(c == cc)
            def _():
                _stripe_copy(acc, out_hbm, s, rows, base_dst=pb)

    return k(dstL, ones, zeros)


def _sc_agg(y, srcL, dstL):
    @functools.partial(
        pl.kernel,
        out_type=jax.ShapeDtypeStruct((_NC * _N, 128), jnp.float32),
        mesh=_sc_mesh(),
        scratch_types=[
            pltpu.VMEM((_NBB, _B), jnp.int32),
            pltpu.VMEM((_NBB, _B), jnp.int32),
            pltpu.VMEM_SHARED((_AR, 128), jnp.float32),
        ] + [pltpu.VMEM((_B, 128), jnp.float32) for _ in range(3)]
          + [pltpu.SemaphoreType.DMA] * 6,
    )
    def k(y_hbm, srcL_hbm, dstL_hbm, out_hbm, src_v, dst_v, acc, *rest):
        g = rest[:3]
        semG = rest[3:6]
        semS = rest[6:9]
        c = lax.axis_index("c")
        s = lax.axis_index("s")
        for p, (pb, rows) in enumerate(_SPANS):
            base = c * _N + pb
            _stripe_copy(y_hbm, acc, s, rows, base_src=base)
            pltpu.sync_copy(srcL_hbm.at[c].at[p].at[s], src_v)
            pltpu.sync_copy(dstL_hbm.at[c].at[p].at[s], dst_v)
            plsc.subcore_barrier()

            for k_ in range(3):
                pltpu.async_copy(y_hbm.at[src_v.at[k_]], g[k_], semG[k_])

            @pl.loop(0, _NBB, step=3)
            def _(j):
                for k_ in range(3):
                    pltpu.make_async_copy(
                        y_hbm.at[src_v.at[j + k_]], g[k_], semG[k_]).wait()
                    pltpu.async_copy(
                        g[k_], acc.at[dst_v.at[j + k_]], semS[k_], add=True)

                for k_ in range(3):
                    @pl.when(j + 3 + k_ < _NBB)
                    def _():
                        pltpu.make_async_copy(
                            g[k_], acc.at[dst_v.at[0]], semS[k_]).wait()
                        pltpu.async_copy(
                            y_hbm.at[src_v.at[j + 3 + k_]], g[k_], semG[k_])

            for k_ in range(3):
                pltpu.make_async_copy(g[k_], acc.at[dst_v.at[0]], semS[k_]).wait()

            plsc.subcore_barrier()
            _stripe_copy(acc, out_hbm, s, rows, base_dst=base)

    return k(y, srcL, dstL)


def _dinv_of(hist_blk):
    deg = jnp.sum(hist_blk, axis=1) * (1.0 / 128.0) + 1.0
    return lax.rsqrt(deg)[:, None]


def _tc1_body(hist_ref, x_ref, w_ref, out_ref):
    dinv = _dinv_of(hist_ref[...])
    y = jnp.dot(x_ref[...], w_ref[...], preferred_element_type=jnp.float32) * dinv
    out_ref[0] = y[:, :128]
    out_ref[1] = y[:, 128:]


def _tc2_body(hist_ref, a_ref, b_ref, w_ref, out_ref):
    dinv = _dinv_of(hist_ref[...])
    a = jnp.concatenate([a_ref[0], a_ref[1]], axis=1)
    h = jnp.maximum(a * dinv + b_ref[...], 0.0)
    y = jnp.dot(h, w_ref[...], preferred_element_type=jnp.float32) * dinv
    out_ref[0] = y[:, :128]
    out_ref[1] = y[:, 128:]


def _tc3_body(hist_ref, a_ref, b2_ref, wl_ref, bl_ref, out_ref):
    dinv = _dinv_of(hist_ref[...])
    a = jnp.concatenate([a_ref[0], a_ref[1]], axis=1)
    h = jnp.maximum(a * dinv + b2_ref[...], 0.0)
    out_ref[...] = (
        jnp.dot(h, wl_ref[...], preferred_element_type=jnp.float32) + bl_ref[...]
    )


_HIST_SPEC = pl.BlockSpec((_MB, 128), lambda i: (i, 0))
_HALF_SPEC = pl.BlockSpec((_NC, _MB, 128), lambda i: (0, i, 0))


def _tc1(hist, x, W1):
    return pl.pallas_call(
        _tc1_body,
        grid=(_GRID,),
        in_specs=[
            _HIST_SPEC,
            pl.BlockSpec((_MB, 128), lambda i: (i, 0)),
            pl.BlockSpec((128, 256), lambda i: (0, 0)),
        ],
        out_specs=_HALF_SPEC,
        out_shape=jax.ShapeDtypeStruct((_NC, _N, 128), jnp.float32),
    )(hist, x, W1)


def _tc2(hist, agg, b1, W2):
    return pl.pallas_call(
        _tc2_body,
        grid=(_GRID,),
        in_specs=[
            _HIST_SPEC,
            _HALF_SPEC,
            pl.BlockSpec((1, 256), lambda i: (0, 0)),
            pl.BlockSpec((256, 256), lambda i: (0, 0)),
        ],
        out_specs=_HALF_SPEC,
        out_shape=jax.ShapeDtypeStruct((_NC, _N, 128), jnp.float32),
    )(hist, agg, b1, W2)


def _tc3(hist, agg, b2, Wl, bl):
    return pl.pallas_call(
        _tc3_body,
        grid=(_GRID,),
        in_specs=[
            _HIST_SPEC,
            _HALF_SPEC,
            pl.BlockSpec((1, 256), lambda i: (0, 0)),
            pl.BlockSpec((256, 128), lambda i: (0, 0)),
            pl.BlockSpec((1, 128), lambda i: (0, 0)),
        ],
        out_specs=pl.BlockSpec((_MB, 128), lambda i: (i, 0)),
        out_shape=jax.ShapeDtypeStruct((_N, 128), jnp.float32),
    )(hist, agg, b2, Wl, bl)


def kernel(x, edge_index, W1, b1, W2, b2, Wl, bl):
    src = edge_index[0].astype(jnp.int32)
    dst = edge_index[1].astype(jnp.int32)
    pad = _EP - _E
    src = jnp.concatenate([src, jnp.zeros((pad,), jnp.int32)])
    dst = jnp.concatenate([dst, jnp.full((pad,), _N, jnp.int32)])
    srcr = src.reshape(_NS, _NBA, _B)
    dstr = dst.reshape(_NS, _NBA, _B)
    zero_l = jnp.arange(_CAP, dtype=jnp.int32) % _N
    trash_l = jnp.full((_CAP,), _TRASH, jnp.int32)
    ones = jnp.ones((_B, 128), jnp.float32)
    zeros = jnp.zeros((_AR, 128), jnp.float32)

    srcL, dstL = _sc_bucket(srcr, dstr, zero_l, trash_l)
    srcL = srcL.reshape(_NC, _NP, _NS, _NBB, _B)
    dstL = dstL.reshape(_NC, _NP, _NS, _NBB, _B)

    hist = _sc_hist(dstL, ones, zeros)
    y1 = _tc1(hist, x, W1)
    agg1 = _sc_agg(y1.reshape(_NC * _N, 128), srcL, dstL).reshape(_NC, _N, 128)
    y2 = _tc2(hist, agg1, b1.reshape(1, 256), W2)
    agg2 = _sc_agg(y2.reshape(_NC * _N, 128), srcL, dstL).reshape(_NC, _N, 128)
    return _tc3(hist, agg2, b2.reshape(1, 256), Wl, bl.reshape(1, 128))


# speedup vs baseline: 10.5090x; 1.0560x over previous
import functools

import jax
import jax.numpy as jnp
from jax import lax
from jax.experimental import pallas as pl
from jax.experimental.pallas import tpu as pltpu
from jax.experimental.pallas import tpu_sc as plsc

_N = 10000
_E = 320000
_NC = 2
_NP = 2
_NS = 16
_SPANS = ((0, 4992), (4992, 5008))
_AR = 5024
_TRASH = _AR - 1
_B = 128
_NBA = 160
_EP = _NS * _NBA * _B
_NBB = 84
_CAP = _NBB * _B
_RING = 4
_RPS = 304
_MB = 1000
_GRID = _N // _MB


def _sc_mesh():
    return plsc.VectorSubcoreMesh(core_axis_name="c", subcore_axis_name="s")


def _stripe_copy(src, dst, s, rows, base_src=0, base_dst=0):
    row0 = s * _RPS
    pltpu.sync_copy(src.at[pl.ds(base_src + row0, _RPS)],
                    dst.at[pl.ds(base_dst + row0, _RPS)])
    tail0 = _RPS * _NS
    tail = rows - tail0

    @pl.when(s == _NS - 1)
    def _():
        pltpu.sync_copy(src.at[pl.ds(base_src + tail0, tail)],
                        dst.at[pl.ds(base_dst + tail0, tail)])


def _sc_bucket(src, dst, zero_l, trash_l):
    @functools.partial(
        pl.kernel,
        out_type=(
            jax.ShapeDtypeStruct((_NC, _NP, _NS, _CAP), jnp.int32),
            jax.ShapeDtypeStruct((_NC, _NP, _NS, _CAP), jnp.int32),
        ),
        mesh=_sc_mesh(),
        compiler_params=pltpu.CompilerParams(needs_layout_passes=False),
        scratch_types=[
            pltpu.VMEM((_NBA, _B), jnp.int32),
            pltpu.VMEM((_NBA, _B), jnp.int32),
            pltpu.VMEM((_CAP,), jnp.int32),
            pltpu.VMEM((_CAP,), jnp.int32),
            pltpu.VMEM((_CAP,), jnp.int32),
            pltpu.VMEM((_CAP,), jnp.int32),
        ],
    )
    def k(src_hbm, dst_hbm, zero_hbm, trash_hbm, srcL_hbm, dstL_hbm,
          src_v, dst_v, sA, dA, sB, dB):
        c = lax.axis_index("c")
        s = lax.axis_index("s")
        off = c * _N
        pltpu.sync_copy(src_hbm.at[s], src_v)
        pltpu.sync_copy(dst_hbm.at[s], dst_v)
        pltpu.sync_copy(zero_hbm, sA)
        pltpu.sync_copy(zero_hbm, sB)
        pltpu.sync_copy(trash_hbm, dA)
        pltpu.sync_copy(trash_hbm, dB)

        def chunk(i, carry):
            c0, c1 = carry
            r = i // 8
            kk = (i % 8) * 16
            dv = dst_v[r, pl.ds(kk, 16)]
            sv = src_v[r, pl.ds(kk, 16)] + off
            m0 = dv < _SPANS[0][1]
            cs0 = plsc.cumsum(m0.astype(jnp.int32))
            pos0 = c0 + cs0 - 1
            ok0 = m0 & (pos0 < _CAP)
            plsc.store_scatter(sA, [pos0], sv, mask=ok0)
            plsc.store_scatter(dA, [pos0], dv, mask=ok0)
            m1 = (dv >= _SPANS[1][0]) & (dv < _N)
            cs1 = plsc.cumsum(m1.astype(jnp.int32))
            pos1 = c1 + cs1 - 1
            ok1 = m1 & (pos1 < _CAP)
            plsc.store_scatter(sB, [pos1], sv, mask=ok1)
            plsc.store_scatter(dB, [pos1], dv - _SPANS[1][0], mask=ok1)
            return (c0 + jnp.max(cs0), c1 + jnp.max(cs1))

        lax.fori_loop(0, _NBA * 8, chunk, (0, 0))

        pltpu.sync_copy(sA, srcL_hbm.at[c].at[0].at[s])
        pltpu.sync_copy(dA, dstL_hbm.at[c].at[0].at[s])
        pltpu.sync_copy(sB, srcL_hbm.at[c].at[1].at[s])
        pltpu.sync_copy(dB, dstL_hbm.at[c].at[1].at[s])

    return k(src, dst, zero_l, trash_l)


def _sc_hist(dstL, ones, zeros):
    @functools.partial(
        pl.kernel,
        out_type=jax.ShapeDtypeStruct((_N, 128), jnp.float32),
        mesh=_sc_mesh(),
        scratch_types=[
            pltpu.VMEM((_NBB, _B), jnp.int32),
            pltpu.VMEM((_B, 128), jnp.float32),
            pltpu.VMEM_SHARED((_AR, 128), jnp.float32),
            pltpu.SemaphoreType.DMA,
        ],
    )
    def k(dstL_hbm, ones_hbm, zeros_hbm, out_hbm, dst_v, ones_v, acc, semS):
        c = lax.axis_index("c")
        s = lax.axis_index("s")
        _stripe_copy(zeros_hbm, acc, s, _AR)
        pltpu.sync_copy(ones_hbm, ones_v)
        pltpu.sync_copy(dstL_hbm.at[c].at[c].at[s], dst_v)
        plsc.subcore_barrier()

        @pl.loop(0, _NBB)
        def _(j):
            @pl.when(j >= 8)
            def _():
                pltpu.make_async_copy(ones_v, acc.at[dst_v.at[0]], semS).wait()

            pltpu.async_copy(ones_v, acc.at[dst_v.at[j]], semS, add=True)

        for _i in range(8):
            pltpu.make_async_copy(ones_v, acc.at[dst_v.at[0]], semS).wait()

        plsc.subcore_barrier()
        for cc, (pb, rows) in enumerate(_SPANS):
            @pl.when(c == cc)
            def _():
                _stripe_copy(acc, out_hbm, s, rows, base_dst=pb)

    return k(dstL, ones, zeros)


def _sc_agg(y, srcL, dstL):
    @functools.partial(
        pl.kernel,
        out_type=jax.ShapeDtypeStruct((_NC * _N, 128), jnp.float32),
        mesh=_sc_mesh(),
        scratch_types=[
            pltpu.VMEM((_NBB, _B), jnp.int32),
            pltpu.VMEM((_NBB, _B), jnp.int32),
            pltpu.VMEM_SHARED((_AR, 128), jnp.float32),
        ] + [pltpu.VMEM((_B, 128), jnp.float32) for _ in range(_RING)]
          + [pltpu.SemaphoreType.DMA] * (2 * _RING),
    )
    def k(y_hbm, srcL_hbm, dstL_hbm, out_hbm, src_v, dst_v, acc, *rest):
        g = rest[:_RING]
        semG = rest[_RING:2 * _RING]
        semS = rest[2 * _RING:3 * _RING]
        c = lax.axis_index("c")
        s = lax.axis_index("s")
        for p, (pb, rows) in enumerate(_SPANS):
            base = c * _N + pb
            _stripe_copy(y_hbm, acc, s, rows, base_src=base)
            pltpu.sync_copy(srcL_hbm.at[c].at[p].at[s], src_v)
            pltpu.sync_copy(dstL_hbm.at[c].at[p].at[s], dst_v)
            plsc.subcore_barrier()

            for k_ in range(_RING):
                pltpu.async_copy(y_hbm.at[src_v.at[k_]], g[k_], semG[k_])

            @pl.loop(0, _NBB, step=_RING)
            def _(j):
                for k_ in range(_RING):
                    pltpu.make_async_copy(
                        y_hbm.at[src_v.at[j + k_]], g[k_], semG[k_]).wait()
                    pltpu.async_copy(
                        g[k_], acc.at[dst_v.at[j + k_]], semS[k_], add=True)

                for k_ in range(_RING):
                    @pl.when(j + _RING + k_ < _NBB)
                    def _():
                        pltpu.make_async_copy(
                            g[k_], acc.at[dst_v.at[0]], semS[k_]).wait()
                        pltpu.async_copy(
                            y_hbm.at[src_v.at[j + _RING + k_]], g[k_], semG[k_])

            for k_ in range(_RING):
                pltpu.make_async_copy(g[k_], acc.at[dst_v.at[0]], semS[k_]).wait()

            plsc.subcore_barrier()
            _stripe_copy(acc, out_hbm, s, rows, base_dst=base)

    return k(y, srcL, dstL)


def _dinv_of(hist_blk):
    deg = jnp.sum(hist_blk, axis=1) * (1.0 / 128.0) + 1.0
    return lax.rsqrt(deg)[:, None]


def _tc1_body(hist_ref, x_ref, w_ref, out_ref):
    dinv = _dinv_of(hist_ref[...])
    y = jnp.dot(x_ref[...], w_ref[...], preferred_element_type=jnp.float32) * dinv
    out_ref[0] = y[:, :128]
    out_ref[1] = y[:, 128:]


def _tc2_body(hist_ref, a_ref, b_ref, w_ref, out_ref):
    dinv = _dinv_of(hist_ref[...])
    a = jnp.concatenate([a_ref[0], a_ref[1]], axis=1)
    h = jnp.maximum(a * dinv + b_ref[...], 0.0)
    y = jnp.dot(h, w_ref[...], preferred_element_type=jnp.float32) * dinv
    out_ref[0] = y[:, :128]
    out_ref[1] = y[:, 128:]


def _tc3_body(hist_ref, a_ref, b2_ref, wl_ref, bl_ref, out_ref):
    dinv = _dinv_of(hist_ref[...])
    a = jnp.concatenate([a_ref[0], a_ref[1]], axis=1)
    h = jnp.maximum(a * dinv + b2_ref[...], 0.0)
    out_ref[...] = (
        jnp.dot(h, wl_ref[...], preferred_element_type=jnp.float32) + bl_ref[...]
    )


_HIST_SPEC = pl.BlockSpec((_MB, 128), lambda i: (i, 0))
_HALF_SPEC = pl.BlockSpec((_NC, _MB, 128), lambda i: (0, i, 0))


def _tc1(hist, x, W1):
    return pl.pallas_call(
        _tc1_body,
        grid=(_GRID,),
        in_specs=[
            _HIST_SPEC,
            pl.BlockSpec((_MB, 128), lambda i: (i, 0)),
            pl.BlockSpec((128, 256), lambda i: (0, 0)),
        ],
        out_specs=_HALF_SPEC,
        out_shape=jax.ShapeDtypeStruct((_NC, _N, 128), jnp.float32),
    )(hist, x, W1)


def _tc2(hist, agg, b1, W2):
    return pl.pallas_call(
        _tc2_body,
        grid=(_GRID,),
        in_specs=[
            _HIST_SPEC,
            _HALF_SPEC,
            pl.BlockSpec((1, 256), lambda i: (0, 0)),
            pl.BlockSpec((256, 256), lambda i: (0, 0)),
        ],
        out_specs=_HALF_SPEC,
        out_shape=jax.ShapeDtypeStruct((_NC, _N, 128), jnp.float32),
    )(hist, agg, b1, W2)


def _tc3(hist, agg, b2, Wl, bl):
    return pl.pallas_call(
        _tc3_body,
        grid=(_GRID,),
        in_specs=[
            _HIST_SPEC,
            _HALF_SPEC,
            pl.BlockSpec((1, 256), lambda i: (0, 0)),
            pl.BlockSpec((256, 128), lambda i: (0, 0)),
            pl.BlockSpec((1, 128), lambda i: (0, 0)),
        ],
        out_specs=pl.BlockSpec((_MB, 128), lambda i: (i, 0)),
        out_shape=jax.ShapeDtypeStruct((_N, 128), jnp.float32),
    )(hist, agg, b2, Wl, bl)


def kernel(x, edge_index, W1, b1, W2, b2, Wl, bl):
    src = edge_index[0].astype(jnp.int32)
    dst = edge_index[1].astype(jnp.int32)
    pad = _EP - _E
    src = jnp.concatenate([src, jnp.zeros((pad,), jnp.int32)])
    dst = jnp.concatenate([dst, jnp.full((pad,), _N, jnp.int32)])
    srcr = src.reshape(_NS, _NBA, _B)
    dstr = dst.reshape(_NS, _NBA, _B)
    zero_l = jnp.arange(_CAP, dtype=jnp.int32) % _N
    trash_l = jnp.full((_CAP,), _TRASH, jnp.int32)
    ones = jnp.ones((_B, 128), jnp.float32)
    zeros = jnp.zeros((_AR, 128), jnp.float32)

    srcL, dstL = _sc_bucket(srcr, dstr, zero_l, trash_l)
    srcL = srcL.reshape(_NC, _NP, _NS, _NBB, _B)
    dstL = dstL.reshape(_NC, _NP, _NS, _NBB, _B)

    hist = _sc_hist(dstL, ones, zeros)
    y1 = _tc1(hist, x, W1)
    agg1 = _sc_agg(y1.reshape(_NC * _N, 128), srcL, dstL).reshape(_NC, _N, 128)
    y2 = _tc2(hist, agg1, b1.reshape(1, 256), W2)
    agg2 = _sc_agg(y2.reshape(_NC * _N, 128), srcL, dstL).reshape(_NC, _N, 128)
    return _tc3(hist, agg2, b2.reshape(1, 256), Wl, bl.reshape(1, 128))
